# baseline (speedup 1.0000x reference)
import jax
import jax.numpy as jnp
from jax.experimental import pallas as pl
from jax.experimental.pallas import tpu as pltpu

B, T, D = 16, 2048, 128
H = 256
E = 128
LSTM = 128
NSH = 4
NEG = -1e30

BB = 8
NSTEP = B // BB
BT = B * T


def _encoder_kernel(state_ref, len2d_ref, w1_ref, b1_ref, w2_ref,
                    b2_ref, wcat_ref, bg_ref, out_ref, emb_ref):
    i = pl.program_id(0)

    x = state_ref[:].reshape(BB * T, D).astype(jnp.bfloat16)
    h = jnp.dot(x, w1_ref[:], preferred_element_type=jnp.float32) + b1_ref[:]
    h = jnp.maximum(h, 0.0).astype(jnp.bfloat16)
    e = jnp.dot(h, w2_ref[:], preferred_element_type=jnp.float32) + b2_ref[:]
    emb_ref[pl.ds(i * BB * T, BB * T), :] = e.astype(jnp.bfloat16)

    @pl.when(i == NSTEP - 1)
    def _pool():
        emb = emb_ref[:]
        j = jax.lax.broadcasted_iota(jnp.int32, (B, BT), 1)
        base = jax.lax.broadcasted_iota(jnp.int32, (B, BT), 0) * T
        t_rel = j - base
        valid = jnp.logical_and(t_rel >= 0, t_rel < len2d_ref[:])
        addend = jnp.where(valid, 0.0, NEG)
        len_f = len2d_ref[:].astype(jnp.float32)
        ct = jnp.zeros((B, LSTM), jnp.float32)
        qt = jnp.zeros((B, LSTM), jnp.float32)
        attended = jax.lax.dot_general(
            valid.astype(jnp.bfloat16), emb, (((1,), (0,)), ((), ())),
            preferred_element_type=jnp.float32) / len_f
        for it in range(NSH):
            gates = jnp.dot(
                jnp.concatenate([attended, qt], axis=1).astype(jnp.bfloat16),
                wcat_ref[:], preferred_element_type=jnp.float32) + bg_ref[:]
            i_g = jax.nn.sigmoid(gates[:, :LSTM])
            f_g = jax.nn.sigmoid(gates[:, LSTM:2 * LSTM])
            g_g = jnp.tanh(gates[:, 2 * LSTM:3 * LSTM])
            o_g = jax.nn.sigmoid(gates[:, 3 * LSTM:])
            ct = f_g * ct + i_g * g_g
            qt = o_g * jnp.tanh(ct)
            if it == NSH - 1:
                break
            logit = jax.lax.dot_general(
                qt.astype(jnp.bfloat16), emb, (((1,), (1,)), ((), ())),
                preferred_element_type=jnp.float32) + addend
            m = jnp.max(logit, axis=1, keepdims=True)
            w = jnp.exp(logit - m)
            s = jnp.sum(w, axis=1, keepdims=True)
            attended = jax.lax.dot_general(
                w.astype(jnp.bfloat16), emb, (((1,), (0,)), ((), ())),
                preferred_element_type=jnp.float32) / s
        out_ref[:, :E] = attended
        out_ref[:, E:] = qt


def _state_imap(i):
    return (i, 0, 0)


def _full(i):
    return (0, 0)


def kernel(state, length, W1, b1, W2, b2, W_ih, W_hh, b_ih, b_hh):
    length = length.astype(jnp.int32)
    len2d = length.reshape(B, 1)
    wcat = jnp.concatenate([W_ih.T, W_hh.T], axis=0)
    return pl.pallas_call(
        _encoder_kernel,
        grid=(NSTEP,),
        in_specs=[
            pl.BlockSpec((BB, T, D), _state_imap),
            pl.BlockSpec((B, 1), _full),
            pl.BlockSpec((D, H), _full),
            pl.BlockSpec((1, H), _full),
            pl.BlockSpec((H, E), _full),
            pl.BlockSpec((1, E), _full),
            pl.BlockSpec((E + LSTM, 4 * LSTM), _full),
            pl.BlockSpec((1, 4 * LSTM), _full),
        ],
        out_specs=pl.BlockSpec((B, E + LSTM), _full),
        out_shape=jax.ShapeDtypeStruct((B, E + LSTM), jnp.float32),
        scratch_shapes=[pltpu.VMEM((BT, E), jnp.bfloat16)],
        compiler_params=pltpu.CompilerParams(
            dimension_semantics=("arbitrary",)),
    )(state, len2d, W1.T.astype(jnp.bfloat16), b1.reshape(1, H),
      W2.T.astype(jnp.bfloat16), b2.reshape(1, E),
      wcat.astype(jnp.bfloat16), (b_ih + b_hh).reshape(1, 4 * LSTM))


# speedup vs baseline: 1.0196x; 1.0196x over previous
import jax
import jax.numpy as jnp
from jax.experimental import pallas as pl
from jax.experimental.pallas import tpu as pltpu

B, T, D = 16, 2048, 128
H = 256
E = 128
LSTM = 128
NSH = 4
NEG = -1e30

BB = 4
NSTEP = B // BB
BT = B * T


def _encoder_kernel(state_ref, len2d_ref, w1_ref, b1_ref, w2_ref,
                    b2_ref, wcat_ref, bg_ref, out_ref, emb_ref):
    i = pl.program_id(0)

    x = state_ref[:].reshape(BB * T, D).astype(jnp.bfloat16)
    h = jnp.dot(x, w1_ref[:], preferred_element_type=jnp.float32) + b1_ref[:]
    h = jnp.maximum(h, 0.0).astype(jnp.bfloat16)
    e = jnp.dot(h, w2_ref[:], preferred_element_type=jnp.float32) + b2_ref[:]
    emb_ref[pl.ds(i * BB * T, BB * T), :] = e.astype(jnp.bfloat16)

    @pl.when(i == NSTEP - 1)
    def _pool():
        emb = emb_ref[:]
        j = jax.lax.broadcasted_iota(jnp.int32, (B, BT), 1)
        base = jax.lax.broadcasted_iota(jnp.int32, (B, BT), 0) * T
        t_rel = j - base
        valid = jnp.logical_and(t_rel >= 0, t_rel < len2d_ref[:])
        addend = jnp.where(valid, 0.0, NEG)
        len_f = len2d_ref[:].astype(jnp.float32)
        ct = jnp.zeros((B, LSTM), jnp.float32)
        qt = jnp.zeros((B, LSTM), jnp.float32)
        attended = jax.lax.dot_general(
            valid.astype(jnp.bfloat16), emb, (((1,), (0,)), ((), ())),
            preferred_element_type=jnp.float32) / len_f
        for it in range(NSH):
            gates = jnp.dot(
                jnp.concatenate([attended, qt], axis=1).astype(jnp.bfloat16),
                wcat_ref[:], preferred_element_type=jnp.float32) + bg_ref[:]
            i_g = jax.nn.sigmoid(gates[:, :LSTM])
            f_g = jax.nn.sigmoid(gates[:, LSTM:2 * LSTM])
            g_g = jnp.tanh(gates[:, 2 * LSTM:3 * LSTM])
            o_g = jax.nn.sigmoid(gates[:, 3 * LSTM:])
            ct = f_g * ct + i_g * g_g
            qt = o_g * jnp.tanh(ct)
            if it == NSH - 1:
                break
            logit = jax.lax.dot_general(
                qt.astype(jnp.bfloat16), emb, (((1,), (1,)), ((), ())),
                preferred_element_type=jnp.float32) + addend
            m = jnp.max(logit, axis=1, keepdims=True)
            w = jnp.exp(logit - m)
            s = jnp.sum(w, axis=1, keepdims=True)
            attended = jax.lax.dot_general(
                w.astype(jnp.bfloat16), emb, (((1,), (0,)), ((), ())),
                preferred_element_type=jnp.float32) / s
        out_ref[:, :E] = attended
        out_ref[:, E:] = qt


def _state_imap(i):
    return (i, 0, 0)


def _full(i):
    return (0, 0)


def kernel(state, length, W1, b1, W2, b2, W_ih, W_hh, b_ih, b_hh):
    length = length.astype(jnp.int32)
    len2d = length.reshape(B, 1)
    wcat = jnp.concatenate([W_ih.T, W_hh.T], axis=0)
    return pl.pallas_call(
        _encoder_kernel,
        grid=(NSTEP,),
        in_specs=[
            pl.BlockSpec((BB, T, D), _state_imap),
            pl.BlockSpec((B, 1), _full),
            pl.BlockSpec((D, H), _full),
            pl.BlockSpec((1, H), _full),
            pl.BlockSpec((H, E), _full),
            pl.BlockSpec((1, E), _full),
            pl.BlockSpec((E + LSTM, 4 * LSTM), _full),
            pl.BlockSpec((1, 4 * LSTM), _full),
        ],
        out_specs=pl.BlockSpec((B, E + LSTM), _full),
        out_shape=jax.ShapeDtypeStruct((B, E + LSTM), jnp.float32),
        scratch_shapes=[pltpu.VMEM((BT, E), jnp.bfloat16)],
        compiler_params=pltpu.CompilerParams(
            dimension_semantics=("arbitrary",)),
    )(state, len2d, W1.T.astype(jnp.bfloat16), b1.reshape(1, H),
      W2.T.astype(jnp.bfloat16), b2.reshape(1, E),
      wcat.astype(jnp.bfloat16), (b_ih + b_hh).reshape(1, 4 * LSTM))
